# baseline (speedup 1.0000x reference)
import functools

import jax
import jax.numpy as jnp
from jax import lax
from jax.experimental import pallas as pl
from jax.experimental.pallas import tpu as pltpu
from jax.experimental.pallas import tpu_sc as plsc

_NC = 2
_NS = 16
_NW = _NC * _NS
_CHUNK = 128
_NBUF = 6
_DEPTH = 3


def _proj_body(n, x_ref, wa_ref, wb_ref, oa_ref, ob_ref):
    x = x_ref[...]
    d = wa_ref.shape[1]
    oa_ref[pl.ds(0, n), pl.ds(0, d)] = jnp.dot(
        x, wa_ref[...], preferred_element_type=jnp.float32)
    ob_ref[pl.ds(0, n), pl.ds(0, d)] = jnp.dot(
        x, wb_ref[...], preferred_element_type=jnp.float32)


def _mid_body(agg_ref, b_ref, wa_ref, wb_ref, oa_ref, ob_ref):
    d = wa_ref.shape[0]
    do = wa_ref.shape[1]
    agg = agg_ref[0, :, pl.ds(0, d)] + agg_ref[1, :, pl.ds(0, d)]
    h = jnp.maximum(agg + b_ref[...], 0.0)
    oa_ref[:, pl.ds(0, do)] = jnp.dot(h, wa_ref[...],
                                      preferred_element_type=jnp.float32)
    ob_ref[:, pl.ds(0, do)] = jnp.dot(h, wb_ref[...],
                                      preferred_element_type=jnp.float32)


def _final_body(n, d, agg_ref, b_ref, o_ref):
    z = (agg_ref[0, pl.ds(0, n), pl.ds(0, d)]
         + agg_ref[1, pl.ds(0, n), pl.ds(0, d)] + b_ref[...])
    m = jnp.max(z, axis=1, keepdims=True)
    s = jnp.sum(jnp.exp(z - m), axis=1, keepdims=True)
    o_ref[...] = (z - m) - jnp.log(s)


def _make_seg_sum(n_pad, n_chunks, d):
    rpt = n_pad // _NS
    base_chunks = n_chunks // _NW
    n_extra = n_chunks - base_chunks * _NW
    assert base_chunks % _NBUF == 0 and base_chunks // _NBUF >= 2
    mesh = plsc.VectorSubcoreMesh(
        core_axis_name="c", subcore_axis_name="s",
        num_cores=_NC, num_subcores=_NS)

    def body(table_hbm, init_hbm, zeros_hbm, edges_hbm, out_hbm,
             src_v, dst_v, rows_v, table_sh, acc_sh, *sems):
        cid = lax.axis_index("c")
        sid = lax.axis_index("s")
        wid = cid * _NS + sid
        row0 = sid * rpt
        @pl.when(cid == 0)
        def _():
            pltpu.sync_copy(init_hbm.at[pl.ds(row0, rpt), pl.ds(0, d)],
                            acc_sh.at[pl.ds(row0, rpt)])
        @pl.when(cid == 1)
        def _():
            pltpu.sync_copy(zeros_hbm.at[pl.ds(row0, rpt)],
                            acc_sh.at[pl.ds(row0, rpt)])
        pltpu.sync_copy(table_hbm.at[pl.ds(row0, rpt), pl.ds(0, d)],
                        table_sh.at[pl.ds(row0, rpt)])
        chunk0 = wid * base_chunks + jnp.minimum(wid, n_extra)
        pltpu.sync_copy(edges_hbm.at[pl.ds(chunk0, base_chunks),
                                     pl.ds(0, _CHUNK)],
                        src_v.at[pl.ds(0, base_chunks)])
        pltpu.sync_copy(edges_hbm.at[pl.ds(chunk0, base_chunks),
                                     pl.ds(_CHUNK, _CHUNK)],
                        dst_v.at[pl.ds(0, base_chunks)])
        @pl.when(wid < n_extra)
        def _():
            pltpu.sync_copy(edges_hbm.at[pl.ds(chunk0 + base_chunks, 1),
                                         pl.ds(0, _CHUNK)],
                            src_v.at[pl.ds(base_chunks, 1)])
            pltpu.sync_copy(edges_hbm.at[pl.ds(chunk0 + base_chunks, 1),
                                         pl.ds(_CHUNK, _CHUNK)],
                            dst_v.at[pl.ds(base_chunks, 1)])
        plsc.subcore_barrier()

        gs = sems[:_NBUF]
        ss = sems[_NBUF:]

        def fire_g(c, b):
            pltpu.async_copy(table_sh.at[src_v.at[c]], rows_v.at[b], gs[b])

        def wait_g(c, b):
            pltpu.make_async_copy(
                table_sh.at[src_v.at[c]], rows_v.at[b], gs[b]).wait()

        def fire_s(c, b):
            pltpu.async_copy(rows_v.at[b], acc_sh.at[dst_v.at[c]], ss[b],
                             add=True)

        def wait_s(c, b):
            pltpu.make_async_copy(
                rows_v.at[b], acc_sh.at[dst_v.at[c]], ss[b]).wait()

        R, D = _NBUF, _DEPTH
        G = base_chunks // R
        for b in range(D):
            fire_g(b, b)
        for c in range(R):
            wait_g(c, c)
            fire_s(c, c)
            if c >= D:
                wait_s(c - D, c - D)
            fire_g(c + D, (c + D) % R)

        def outer(g, carry):
            base = g * R
            for b in range(R):
                c = base + b
                wait_g(c, b)
                fire_s(c, b)
                wait_s(c - D, (b + R - D) % R)
                fire_g(c + D, (b + D) % R)
            return carry
        lax.fori_loop(1, G - 1, outer, 0)

        base = (G - 1) * R
        for b in range(R):
            c = base + b
            wait_g(c, b)
            fire_s(c, b)
            wait_s(c - D, (b + R - D) % R)
            if c + D < base_chunks:
                fire_g(c + D, (b + D) % R)
        for k in range(D):
            c = base_chunks - D + k
            wait_s(c, c % R)

        @pl.when(wid < n_extra)
        def _():
            pltpu.sync_copy(table_sh.at[src_v.at[base_chunks]], rows_v.at[0])
            pltpu.sync_copy(rows_v.at[0], acc_sh.at[dst_v.at[base_chunks]],
                            add=True)

        plsc.subcore_barrier()
        pltpu.sync_copy(acc_sh.at[pl.ds(row0, rpt)],
                        out_hbm.at[cid, pl.ds(row0, rpt), pl.ds(0, d)])

    return pl.kernel(
        body,
        out_type=jax.ShapeDtypeStruct((_NC, n_pad, 128), jnp.float32),
        mesh=mesh,
        scratch_types=[
            pltpu.VMEM((base_chunks + 1, _CHUNK), jnp.int32),
            pltpu.VMEM((base_chunks + 1, _CHUNK), jnp.int32),
            pltpu.VMEM((_NBUF, _CHUNK, d), jnp.float32),
            pltpu.VMEM_SHARED((n_pad, d), jnp.float32),
            pltpu.VMEM_SHARED((n_pad, d), jnp.float32),
        ] + [pltpu.SemaphoreType.DMA] * (2 * _NBUF),
        compiler_params=pltpu.CompilerParams(use_tc_tiling_on_sc=False),
    )


def kernel(x, edge_index, W1_root, W1_neigh, b1, W2_root, W2_neigh, b2):
    n, _ = x.shape
    dh = W1_root.shape[1]
    do = W2_root.shape[1]
    e = edge_index.shape[1]

    n_pad = -(-n // (_NS * 8)) * (_NS * 8)
    assert e % _CHUNK == 0
    n_chunks = e // _CHUNK
    edges = jnp.concatenate(
        [edge_index[0].reshape(n_chunks, _CHUNK),
         edge_index[1].reshape(n_chunks, _CHUNK)], axis=1)

    f32 = jnp.float32
    wide = jax.ShapeDtypeStruct((n_pad, 128), f32)
    xr, xn = pl.pallas_call(
        functools.partial(_proj_body, n),
        out_shape=[wide, wide],
    )(x, W1_root, W1_neigh)

    seg_sum = _make_seg_sum(n_pad, n_chunks, dh)
    zeros = jnp.zeros((n_pad, dh), f32)
    agg1 = seg_sum(xn, xr, zeros, edges)

    hr, hn = pl.pallas_call(
        _mid_body,
        out_shape=[wide, wide],
    )(agg1, b1.reshape(1, dh), W2_root, W2_neigh)

    if do == dh:
        seg_sum2, zeros2 = seg_sum, zeros
    else:
        seg_sum2 = _make_seg_sum(n_pad, n_chunks, do)
        zeros2 = jnp.zeros((n_pad, do), f32)
    agg2 = seg_sum2(hn, hr, zeros2, edges)

    out = pl.pallas_call(
        functools.partial(_final_body, n, do),
        out_shape=jax.ShapeDtypeStruct((n, do), f32),
    )(agg2, b2.reshape(1, do))
    return out


# speedup vs baseline: 1.0342x; 1.0342x over previous
import functools

import jax
import jax.numpy as jnp
from jax import lax
from jax.experimental import pallas as pl
from jax.experimental.pallas import tpu as pltpu
from jax.experimental.pallas import tpu_sc as plsc

_NC = 2
_NS = 16
_NW = _NC * _NS
_CHUNK = 128
_NBUF = 6
_DEPTH = 3


def _proj_body(n, x_ref, wa_ref, wb_ref, oa_ref, ob_ref):
    x = x_ref[...]
    d = wa_ref.shape[1]
    oa_ref[pl.ds(0, n), pl.ds(0, d)] = jnp.dot(
        x, wa_ref[...], preferred_element_type=jnp.float32)
    ob_ref[pl.ds(0, n), pl.ds(0, d)] = jnp.dot(
        x, wb_ref[...], preferred_element_type=jnp.float32)


def _mid_body(agg_ref, b_ref, wa_ref, wb_ref, oa_ref, ob_ref):
    d = wa_ref.shape[0]
    do = wa_ref.shape[1]
    agg = agg_ref[0, :, pl.ds(0, d)] + agg_ref[1, :, pl.ds(0, d)]
    h = jnp.maximum(agg + b_ref[...], 0.0)
    oa_ref[:, pl.ds(0, do)] = jnp.dot(h, wa_ref[...],
                                      preferred_element_type=jnp.float32)
    ob_ref[:, pl.ds(0, do)] = jnp.dot(h, wb_ref[...],
                                      preferred_element_type=jnp.float32)


def _final_body(n, d, agg_ref, b_ref, o_ref):
    z = (agg_ref[0, pl.ds(0, n), pl.ds(0, d)]
         + agg_ref[1, pl.ds(0, n), pl.ds(0, d)] + b_ref[...])
    m = jnp.max(z, axis=1, keepdims=True)
    s = jnp.sum(jnp.exp(z - m), axis=1, keepdims=True)
    o_ref[...] = (z - m) - jnp.log(s)


def _make_seg_sum(n_pad, n_chunks, d):
    rpt = n_pad // _NS
    base_chunks = n_chunks // _NW
    n_extra = n_chunks - base_chunks * _NW
    assert base_chunks % _NBUF == 0 and base_chunks // _NBUF >= 2
    mesh = plsc.VectorSubcoreMesh(
        core_axis_name="c", subcore_axis_name="s",
        num_cores=_NC, num_subcores=_NS)

    def body(table_hbm, init_hbm, zeros_hbm, edges_hbm, out_hbm,
             src_v, dst_v, rows_v, table_sh, acc_sh, *sems):
        cid = lax.axis_index("c")
        sid = lax.axis_index("s")
        wid = cid * _NS + sid
        row0 = sid * rpt
        @pl.when(cid == 0)
        def _():
            pltpu.sync_copy(init_hbm.at[pl.ds(row0, rpt), pl.ds(0, d)],
                            acc_sh.at[pl.ds(row0, rpt)])
        @pl.when(cid == 1)
        def _():
            pltpu.sync_copy(zeros_hbm.at[pl.ds(row0, rpt)],
                            acc_sh.at[pl.ds(row0, rpt)])
        pltpu.sync_copy(table_hbm.at[pl.ds(row0, rpt), pl.ds(0, d)],
                        table_sh.at[pl.ds(row0, rpt)])
        chunk0 = wid * base_chunks + jnp.minimum(wid, n_extra)
        pltpu.sync_copy(edges_hbm.at[pl.ds(chunk0, base_chunks),
                                     pl.ds(0, _CHUNK)],
                        src_v.at[pl.ds(0, base_chunks)])
        pltpu.sync_copy(edges_hbm.at[pl.ds(chunk0, base_chunks),
                                     pl.ds(_CHUNK, _CHUNK)],
                        dst_v.at[pl.ds(0, base_chunks)])
        @pl.when(wid < n_extra)
        def _():
            pltpu.sync_copy(edges_hbm.at[pl.ds(chunk0 + base_chunks, 1),
                                         pl.ds(0, _CHUNK)],
                            src_v.at[pl.ds(base_chunks, 1)])
            pltpu.sync_copy(edges_hbm.at[pl.ds(chunk0 + base_chunks, 1),
                                         pl.ds(_CHUNK, _CHUNK)],
                            dst_v.at[pl.ds(base_chunks, 1)])
        plsc.subcore_barrier()

        gs = sems[:_NBUF]
        ss = sems[_NBUF:]

        def fire_g(c, b):
            pltpu.async_copy(table_sh.at[src_v.at[c]], rows_v.at[b], gs[b])

        def wait_g(c, b):
            pltpu.make_async_copy(
                table_sh.at[src_v.at[c]], rows_v.at[b], gs[b]).wait()

        def fire_s(c, b):
            pltpu.async_copy(rows_v.at[b], acc_sh.at[dst_v.at[c]], ss[b],
                             add=True)

        def wait_s(c, b):
            pltpu.make_async_copy(
                rows_v.at[b], acc_sh.at[dst_v.at[c]], ss[b]).wait()

        R, D = _NBUF, _DEPTH
        G = base_chunks // R
        for b in range(D):
            fire_g(b, b)
        for c in range(R):
            wait_g(c, c)
            fire_s(c, c)
            if c >= D:
                wait_s(c - D, c - D)
            fire_g(c + D, (c + D) % R)

        def outer(g, carry):
            base = g * R
            for b in range(R):
                c = base + b
                wait_g(c, b)
                fire_s(c, b)
                wait_s(c - D, (b + R - D) % R)
                fire_g(c + D, (b + D) % R)
            return carry
        lax.fori_loop(1, G - 1, outer, 0)

        base = (G - 1) * R
        for b in range(R):
            c = base + b
            wait_g(c, b)
            fire_s(c, b)
            wait_s(c - D, (b + R - D) % R)
            if c + D < base_chunks:
                fire_g(c + D, (b + D) % R)
        for k in range(D):
            c = base_chunks - D + k
            wait_s(c, c % R)

        @pl.when(wid < n_extra)
        def _():
            pltpu.sync_copy(table_sh.at[src_v.at[base_chunks]], rows_v.at[0])
            pltpu.sync_copy(rows_v.at[0], acc_sh.at[dst_v.at[base_chunks]],
                            add=True)

        plsc.subcore_barrier()
        pltpu.sync_copy(acc_sh.at[pl.ds(row0, rpt)],
                        out_hbm.at[cid, pl.ds(row0, rpt), pl.ds(0, d)])

    return pl.kernel(
        body,
        out_type=jax.ShapeDtypeStruct((_NC, n_pad, 128), jnp.float32),
        mesh=mesh,
        scratch_types=[
            pltpu.VMEM((base_chunks + 1, _CHUNK), jnp.int32),
            pltpu.VMEM((base_chunks + 1, _CHUNK), jnp.int32),
            pltpu.VMEM((_NBUF, _CHUNK, d), jnp.float32),
            pltpu.VMEM_SHARED((n_pad, d), jnp.float32),
            pltpu.VMEM_SHARED((n_pad, d), jnp.float32),
        ] + [pltpu.SemaphoreType.DMA] * (2 * _NBUF),
        compiler_params=pltpu.CompilerParams(use_tc_tiling_on_sc=False),
    )


def kernel(x, edge_index, W1_root, W1_neigh, b1, W2_root, W2_neigh, b2):
    n, _ = x.shape
    dh = W1_root.shape[1]
    do = W2_root.shape[1]
    e = edge_index.shape[1]

    n_pad = -(-n // (_NS * 8)) * (_NS * 8)
    assert e % _CHUNK == 0
    n_chunks = e // _CHUNK
    edges = edge_index.reshape(2, n_chunks, _CHUNK).transpose(1, 0, 2) \
                      .reshape(n_chunks, 2 * _CHUNK)

    f32 = jnp.float32
    wide = jax.ShapeDtypeStruct((n_pad, 128), f32)
    xr, xn = pl.pallas_call(
        functools.partial(_proj_body, n),
        out_shape=[wide, wide],
    )(x, W1_root, W1_neigh)

    seg_sum = _make_seg_sum(n_pad, n_chunks, dh)
    zeros = jnp.zeros((n_pad, dh), f32)
    agg1 = seg_sum(xn, xr, zeros, edges)

    hr, hn = pl.pallas_call(
        _mid_body,
        out_shape=[wide, wide],
    )(agg1, b1.reshape(1, dh), W2_root, W2_neigh)

    if do == dh:
        seg_sum2, zeros2 = seg_sum, zeros
    else:
        seg_sum2 = _make_seg_sum(n_pad, n_chunks, do)
        zeros2 = jnp.zeros((n_pad, do), f32)
    agg2 = seg_sum2(hn, hr, zeros2, edges)

    out = pl.pallas_call(
        functools.partial(_final_body, n, do),
        out_shape=jax.ShapeDtypeStruct((n, do), f32),
    )(agg2, b2.reshape(1, do))
    return out
